# baseline (speedup 1.0000x reference)
import functools

import jax
import jax.numpy as jnp
from jax import lax
from jax.experimental import pallas as pl
from jax.experimental.pallas import tpu as pltpu
from jax.experimental.pallas import tpu_sc as plsc

N = 1_000_000
D = 64
N2 = 1 << 20
R = N2 // 128
SUMBLK = 2000


def _sum_body(x_ref, o_ref):
    x = x_ref[...]

    def rollc(v, d):
        return jnp.concatenate([v[:, d:], v[:, :d]], axis=1)

    acc = None
    for c in (0, 1):
        xc = x[:, 32 * c:32 * c + 32]
        t = xc
        for i in (8, 16, 24):
            t = t + jnp.concatenate([xc[:, i:], xc[:, :i]], axis=1)
        for d in (4, 2, 1):
            t = t + jnp.concatenate([t[:, d:], t[:, :d]], axis=1)
        acc = t[:, 0] if acc is None else acc + t[:, 0]
    o_ref[...] = acc.reshape(1, 1, SUMBLK)


def _row_sums(input):
    out = pl.pallas_call(
        _sum_body,
        grid=(N // SUMBLK,),
        in_specs=[pl.BlockSpec((SUMBLK, D), lambda i: (i, 0))],
        out_specs=pl.BlockSpec((1, 1, SUMBLK), lambda i: (i, 0, 0)),
        out_shape=jax.ShapeDtypeStruct((N // SUMBLK, 1, SUMBLK), jnp.float32),
    )(input)
    return out.reshape(N)


def _lex_greater(k, i, kp, ip):
    return (k > kp) | ((k == kp) & (i > ip))


def _roll(x, dl, axis):
    if axis == 0:
        return jnp.concatenate([x[dl:, :], x[:dl, :]], axis=0)
    return jnp.concatenate([x[:, dl:], x[:, :dl]], axis=1)


def _local_exchange(kv, iv, j, ascm, r_io, c_io, enable=None):
    d = 1 << j
    if d >= 128:
        axis, dl, coord = 0, d // 128, r_io
    else:
        axis, dl, coord = 1, d, c_io
    lob = (coord & dl) == 0
    km = _roll(kv, dl, axis)
    kp_ = _roll(kv, -dl, axis)
    im = _roll(iv, dl, axis)
    ip_ = _roll(iv, -dl, axis)
    kpart = jnp.where(lob, km, kp_)
    ipart = jnp.where(lob, im, ip_)
    g = _lex_greater(kv, iv, kpart, ipart)
    take = jnp.logical_xor(jnp.logical_xor(g, lob), ascm)
    if enable is not None:
        take = take & enable
    return jnp.where(take, kpart, kv), jnp.where(take, ipart, iv)


def _local13_body(k_ref, i_ref, ko_ref, io_ref):
    kv = k_ref[...]
    iv = i_ref[...]
    r_io = lax.broadcasted_iota(jnp.int32, (128, 128), 0)
    c_io = lax.broadcasted_iota(jnp.int32, (128, 128), 1)
    lflat = r_io * 128 + c_io
    for lvl in range(1, 14):
        ascm = (lflat & (1 << lvl)) == 0
        for j in range(lvl - 1, -1, -1):
            kv, iv = _local_exchange(kv, iv, j, ascm, r_io, c_io)
    ko_ref[...] = kv
    io_ref[...] = iv


def _localdyn_body(lvl_ref, k_ref, i_ref, ko_ref, io_ref):
    kv = k_ref[...]
    iv = i_ref[...]
    lvl = lvl_ref[0]
    blk = pl.program_id(0)
    r_io = lax.broadcasted_iota(jnp.int32, (128, 128), 0)
    c_io = lax.broadcasted_iota(jnp.int32, (128, 128), 1)
    flat = (blk * 128 + r_io) * 128 + c_io
    ascm = ((flat >> lvl) & 1) == 0
    for j in range(13, -1, -1):
        enable = (j < lvl) & jnp.full((128, 128), True)
        kv, iv = _local_exchange(kv, iv, j, ascm, r_io, c_io, enable)
    ko_ref[...] = kv
    io_ref[...] = iv


def _global_body(m, lvl_ref, k_ref, i_ref, ko_ref, io_ref):
    lvl = lvl_ref[0]
    g = pl.program_id(0)
    ka = k_ref[:m, :]
    kb = k_ref[m:, :]
    ia = i_ref[:m, :]
    ib = i_ref[m:, :]
    base_row = g * (2 * m)
    asc = ((base_row >> (lvl - 7)) & 1) == 0
    gr = _lex_greater(ka, ia, kb, ib)
    take = jnp.logical_xor(gr, jnp.logical_not(asc))
    ko_ref[:m, :] = jnp.where(take, kb, ka)
    ko_ref[m:, :] = jnp.where(take, ka, kb)
    io_ref[:m, :] = jnp.where(take, ib, ia)
    io_ref[m:, :] = jnp.where(take, ia, ib)


def _mk_pair_call(body, rows, grid):
    spec = pl.BlockSpec((rows, 128), lambda g: (g, 0))
    sspec = pl.BlockSpec(memory_space=pltpu.SMEM)
    return pl.pallas_call(
        body,
        grid=(grid,),
        in_specs=[sspec, spec, spec],
        out_specs=[spec, spec],
        out_shape=[
            jax.ShapeDtypeStruct((R, 128), jnp.float32),
            jax.ShapeDtypeStruct((R, 128), jnp.int32),
        ],
        input_output_aliases={1: 0, 2: 1},
    )


_local13_call = None
_localdyn_call = None
_global_calls = {}


def _build_calls():
    global _local13_call, _localdyn_call
    spec = pl.BlockSpec((128, 128), lambda g: (g, 0))
    _local13_call = pl.pallas_call(
        _local13_body,
        grid=(R // 128,),
        in_specs=[spec, spec],
        out_specs=[spec, spec],
        out_shape=[
            jax.ShapeDtypeStruct((R, 128), jnp.float32),
            jax.ShapeDtypeStruct((R, 128), jnp.int32),
        ],
        input_output_aliases={0: 0, 1: 1},
    )
    _localdyn_call = _mk_pair_call(_localdyn_body, 128, R // 128)
    for j in range(14, 20):
        m = 1 << (j - 7)
        _global_calls[j] = _mk_pair_call(
            functools.partial(_global_body, m), 2 * m, R // (2 * m))


_build_calls()


def _bitonic_sort(keys, idx):
    kv, iv = _local13_call(keys, idx)
    for lvl in range(14, 21):
        l_arr = jnp.array([lvl], dtype=jnp.int32)
        for j in range(lvl - 1, 13, -1):
            kv, iv = _global_calls[j](l_arr, kv, iv)
        kv, iv = _localdyn_call(l_arr, kv, iv)
    return kv, iv


_NC = 2
_NS = 16
_NW = _NC * _NS
_CH = 64
_NCHUNK = N // _CH


def _gather_sc(table_hbm, idx_hbm, out_hbm, idx_v, rows_v, sem):
    wid = lax.axis_index("s") * _NC + lax.axis_index("c")
    nch = 488 + jnp.where(wid < _NCHUNK - 488 * _NW, 1, 0)

    def body(g, carry):
        c = wid + _NW * g
        pltpu.sync_copy(idx_hbm.at[c], idx_v)
        pltpu.async_copy(table_hbm.at[idx_v], rows_v, sem).wait()
        pltpu.sync_copy(rows_v, out_hbm.at[pl.ds(c * _CH, _CH)])
        return carry

    lax.fori_loop(0, nch, body, 0)


def _make_gather():
    mesh = plsc.VectorSubcoreMesh(core_axis_name="c", subcore_axis_name="s")
    return functools.partial(
        pl.kernel,
        mesh=mesh,
        out_type=jax.ShapeDtypeStruct((N, 128), jnp.float32),
        scratch_types=[
            pltpu.VMEM((_CH,), jnp.int32),
            pltpu.VMEM((_CH, 128), jnp.float32),
            pltpu.SemaphoreType.DMA,
        ],
    )(_gather_sc)


_gather_call = _make_gather()


def kernel(input):
    s = _row_sums(input)
    keys = jnp.concatenate(
        [-s, jnp.full((N2 - N,), jnp.inf, jnp.float32)]).reshape(R, 128)
    idx = jnp.arange(N2, dtype=jnp.int32).reshape(R, 128)
    _, iv = _bitonic_sort(keys, idx)
    sorted_idx = iv.reshape(-1)[:N].reshape(N // _CH, _CH)
    table = jnp.pad(input, ((0, 0), (0, 128 - D)))
    return _gather_call(table, sorted_idx)[:, :D]


# speedup vs baseline: 1.0011x; 1.0011x over previous
import functools

import jax
import jax.numpy as jnp
from jax import lax
from jax.experimental import pallas as pl
from jax.experimental.pallas import tpu as pltpu
from jax.experimental.pallas import tpu_sc as plsc

N = 1_000_000
D = 64
N2 = 1 << 20
R = N2 // 128
SUMBLK = 2000


def _sum_body(x_ref, o_ref):
    x = x_ref[...]

    def rollc(v, d):
        return jnp.concatenate([v[:, d:], v[:, :d]], axis=1)

    acc = None
    for c in (0, 1):
        xc = x[:, 32 * c:32 * c + 32]
        t = xc
        for i in (8, 16, 24):
            t = t + jnp.concatenate([xc[:, i:], xc[:, :i]], axis=1)
        for d in (4, 2, 1):
            t = t + jnp.concatenate([t[:, d:], t[:, :d]], axis=1)
        acc = t[:, 0] if acc is None else acc + t[:, 0]
    o_ref[...] = acc.reshape(1, 1, SUMBLK)


def _row_sums(input):
    out = pl.pallas_call(
        _sum_body,
        grid=(N // SUMBLK,),
        in_specs=[pl.BlockSpec((SUMBLK, D), lambda i: (i, 0))],
        out_specs=pl.BlockSpec((1, 1, SUMBLK), lambda i: (i, 0, 0)),
        out_shape=jax.ShapeDtypeStruct((N // SUMBLK, 1, SUMBLK), jnp.float32),
    )(input)
    return out.reshape(N)


def _lex_greater(k, i, kp, ip):
    return (k > kp) | ((k == kp) & (i > ip))


def _roll(x, dl, axis):
    if axis == 0:
        return jnp.concatenate([x[dl:, :], x[:dl, :]], axis=0)
    return jnp.concatenate([x[:, dl:], x[:, :dl]], axis=1)


def _local_exchange(kv, iv, j, ascm, r_io, c_io, enable=None):
    d = 1 << j
    if d >= 1024:
        dl = d // 128
        grp = 128 // (2 * dl)
        kr = kv.reshape(grp, 2, dl, 128)
        ir = iv.reshape(grp, 2, dl, 128)
        ka, kb = kr[:, 0], kr[:, 1]
        ia, ib = ir[:, 0], ir[:, 1]
        asc = ascm.reshape(grp, 2, dl, 128)[:, 0]
        gr = _lex_greater(ka, ia, kb, ib)
        take = jnp.logical_xor(gr, jnp.logical_not(asc))
        if enable is not None:
            take = take & enable.reshape(grp, 2, dl, 128)[:, 0]
        ka2 = jnp.where(take, kb, ka)
        kb2 = jnp.where(take, ka, kb)
        ia2 = jnp.where(take, ib, ia)
        ib2 = jnp.where(take, ia, ib)
        kv = jnp.concatenate([ka2[:, None], kb2[:, None]], axis=1).reshape(128, 128)
        iv = jnp.concatenate([ia2[:, None], ib2[:, None]], axis=1).reshape(128, 128)
        return kv, iv
    if d >= 128:
        axis, dl, coord = 0, d // 128, r_io
    else:
        axis, dl, coord = 1, d, c_io
    lob = (coord & dl) == 0
    km = _roll(kv, dl, axis)
    kp_ = _roll(kv, -dl, axis)
    im = _roll(iv, dl, axis)
    ip_ = _roll(iv, -dl, axis)
    kpart = jnp.where(lob, km, kp_)
    ipart = jnp.where(lob, im, ip_)
    g = _lex_greater(kv, iv, kpart, ipart)
    take = jnp.logical_xor(jnp.logical_xor(g, lob), ascm)
    if enable is not None:
        take = take & enable
    return jnp.where(take, kpart, kv), jnp.where(take, ipart, iv)


def _local13_body(k_ref, i_ref, ko_ref, io_ref):
    kv = k_ref[...]
    iv = i_ref[...]
    r_io = lax.broadcasted_iota(jnp.int32, (128, 128), 0)
    c_io = lax.broadcasted_iota(jnp.int32, (128, 128), 1)
    lflat = r_io * 128 + c_io
    for lvl in range(1, 14):
        ascm = (lflat & (1 << lvl)) == 0
        for j in range(lvl - 1, -1, -1):
            kv, iv = _local_exchange(kv, iv, j, ascm, r_io, c_io)
    ko_ref[...] = kv
    io_ref[...] = iv


def _localdyn_body(lvl_ref, k_ref, i_ref, ko_ref, io_ref):
    kv = k_ref[...]
    iv = i_ref[...]
    lvl = lvl_ref[0]
    blk = pl.program_id(0)
    r_io = lax.broadcasted_iota(jnp.int32, (128, 128), 0)
    c_io = lax.broadcasted_iota(jnp.int32, (128, 128), 1)
    flat = (blk * 128 + r_io) * 128 + c_io
    ascm = ((flat >> lvl) & 1) == 0
    for j in range(13, -1, -1):
        enable = (j < lvl) & jnp.full((128, 128), True)
        kv, iv = _local_exchange(kv, iv, j, ascm, r_io, c_io, enable)
    ko_ref[...] = kv
    io_ref[...] = iv


def _global_body(m, lvl_ref, k_ref, i_ref, ko_ref, io_ref):
    lvl = lvl_ref[0]
    g = pl.program_id(0)
    ka = k_ref[:m, :]
    kb = k_ref[m:, :]
    ia = i_ref[:m, :]
    ib = i_ref[m:, :]
    base_row = g * (2 * m)
    asc = ((base_row >> (lvl - 7)) & 1) == 0
    gr = _lex_greater(ka, ia, kb, ib)
    take = jnp.logical_xor(gr, jnp.logical_not(asc))
    ko_ref[:m, :] = jnp.where(take, kb, ka)
    ko_ref[m:, :] = jnp.where(take, ka, kb)
    io_ref[:m, :] = jnp.where(take, ib, ia)
    io_ref[m:, :] = jnp.where(take, ia, ib)


def _mk_pair_call(body, rows, grid):
    spec = pl.BlockSpec((rows, 128), lambda g: (g, 0))
    sspec = pl.BlockSpec(memory_space=pltpu.SMEM)
    return pl.pallas_call(
        body,
        grid=(grid,),
        in_specs=[sspec, spec, spec],
        out_specs=[spec, spec],
        out_shape=[
            jax.ShapeDtypeStruct((R, 128), jnp.float32),
            jax.ShapeDtypeStruct((R, 128), jnp.int32),
        ],
        input_output_aliases={1: 0, 2: 1},
    )


_local13_call = None
_localdyn_call = None
_global_calls = {}


def _build_calls():
    global _local13_call, _localdyn_call
    spec = pl.BlockSpec((128, 128), lambda g: (g, 0))
    _local13_call = pl.pallas_call(
        _local13_body,
        grid=(R // 128,),
        in_specs=[spec, spec],
        out_specs=[spec, spec],
        out_shape=[
            jax.ShapeDtypeStruct((R, 128), jnp.float32),
            jax.ShapeDtypeStruct((R, 128), jnp.int32),
        ],
        input_output_aliases={0: 0, 1: 1},
    )
    _localdyn_call = _mk_pair_call(_localdyn_body, 128, R // 128)
    for j in range(14, 20):
        m = 1 << (j - 7)
        _global_calls[j] = _mk_pair_call(
            functools.partial(_global_body, m), 2 * m, R // (2 * m))


_build_calls()


def _bitonic_sort(keys, idx):
    kv, iv = _local13_call(keys, idx)
    for lvl in range(14, 21):
        l_arr = jnp.array([lvl], dtype=jnp.int32)
        for j in range(lvl - 1, 13, -1):
            kv, iv = _global_calls[j](l_arr, kv, iv)
        kv, iv = _localdyn_call(l_arr, kv, iv)
    return kv, iv


_NC = 2
_NS = 16
_NW = _NC * _NS
_CH = 64
_NCHUNK = N // _CH


def _gather_sc(table_hbm, idx_hbm, out_hbm, idx_v, rows_v, sem):
    wid = lax.axis_index("s") * _NC + lax.axis_index("c")
    nch = 488 + jnp.where(wid < _NCHUNK - 488 * _NW, 1, 0)

    def body(g, carry):
        c = wid + _NW * g
        pltpu.sync_copy(idx_hbm.at[c], idx_v)
        pltpu.async_copy(table_hbm.at[idx_v], rows_v, sem).wait()
        pltpu.sync_copy(rows_v, out_hbm.at[pl.ds(c * _CH, _CH)])
        return carry

    lax.fori_loop(0, nch, body, 0)


def _make_gather():
    mesh = plsc.VectorSubcoreMesh(core_axis_name="c", subcore_axis_name="s")
    return functools.partial(
        pl.kernel,
        mesh=mesh,
        out_type=jax.ShapeDtypeStruct((N, 128), jnp.float32),
        scratch_types=[
            pltpu.VMEM((_CH,), jnp.int32),
            pltpu.VMEM((_CH, 128), jnp.float32),
            pltpu.SemaphoreType.DMA,
        ],
    )(_gather_sc)


_gather_call = _make_gather()


def kernel(input):
    s = _row_sums(input)
    keys = jnp.concatenate(
        [-s, jnp.full((N2 - N,), jnp.inf, jnp.float32)]).reshape(R, 128)
    idx = jnp.arange(N2, dtype=jnp.int32).reshape(R, 128)
    _, iv = _bitonic_sort(keys, idx)
    sorted_idx = iv.reshape(-1)[:N].reshape(N // _CH, _CH)
    table = jnp.pad(input, ((0, 0), (0, 128 - D)))
    return _gather_call(table, sorted_idx)[:, :D]
